# baseline (speedup 1.0000x reference)
import jax
import jax.numpy as jnp
from jax import lax
from jax.experimental import pallas as pl
from jax.experimental.pallas import tpu as pltpu
from jax.experimental.pallas import tpu_sc as plsc

_T = 16384
_H = 2048
_SH = 16
_K = 2
_E = 16
_N = _T * _K
_NC = 2
_NS = 16
_NW = _NC * _NS
_CHUNK = _N // _NW
_TOK = _T // _NW
_BT = 16
_NB = _CHUNK // _BT
_VREGS = _CHUNK // 16


def _index_body(topk_hbm, starts_hbm, xs_hbm, oidx_hbm, outs_hbm,
                hist_ids_v, own_ids_v, dest_v, cnt_v, hist_v, hist_all_v,
                starts_v, sbuf, idx_a_v, idx_b_v, hist_sh, sem):
    c = lax.axis_index("c")
    s = lax.axis_index("s")
    wid = c * _NS + s
    lanes = lax.iota(jnp.int32, 16)

    pltpu.sync_copy(topk_hbm.at[pl.ds(s * 2 * _CHUNK, 2 * _CHUNK)], hist_ids_v)
    for j in range(2):
        def hist_step(r, acc, j=j):
            v = hist_ids_v[pl.ds(j * _CHUNK + r * 16, 16)]
            for e in range(_E):
                tot = jnp.sum((v == e).astype(jnp.int32))
                acc = jnp.where(lanes == e, acc + tot, acc)
            return acc
        hist_v[...] = lax.fori_loop(0, _VREGS, hist_step,
                                    jnp.zeros((16,), jnp.int32))
        pltpu.sync_copy(hist_v, hist_sh.at[pl.ds((2 * s + j) * 16, 16)])

    plsc.subcore_barrier()

    pltpu.sync_copy(hist_sh, hist_all_v)
    pltpu.sync_copy(starts_hbm, starts_v)
    base = lax.fori_loop(
        0, wid, lambda w, acc: acc + hist_all_v[pl.ds(w * 16, 16)],
        starts_v[...])
    cnt_v[...] = base

    pltpu.sync_copy(topk_hbm.at[pl.ds(wid * _CHUNK, _CHUNK)], own_ids_v)

    def rank_step(r, _):
        v = own_ids_v[pl.ds(r * 16, 16)]
        g = plsc.load_gather(cnt_v, [v])
        rank = jnp.zeros((16,), jnp.int32)
        vc = jnp.zeros((16,), jnp.int32)
        for e in range(_E):
            m = v == e
            cum = jnp.cumsum(m.astype(jnp.int32))
            rank = jnp.where(m, cum - 1, rank)
            tot = jnp.sum(m.astype(jnp.int32))
            vc = jnp.where(lanes == e, vc + tot, vc)
        dest_v[pl.ds(r * 16, 16)] = g + rank
        cnt_v[...] = cnt_v[...] + vc
        return 0

    lax.fori_loop(0, _VREGS, rank_step, 0)

    pltpu.sync_copy(dest_v, oidx_hbm.at[pl.ds(wid * _CHUNK, _CHUNK)])

    pltpu.sync_copy(xs_hbm.at[pl.ds(wid * _TOK, _TOK)], sbuf)
    for j in range(4):
        for i in range(8):
            base = 256 * j + 32 * i
            idx_a_v[j, pl.ds(16 * i, 16)] = plsc.load_gather(
                dest_v, [base + 2 * lanes])
            idx_b_v[j, pl.ds(16 * i, 16)] = plsc.load_gather(
                dest_v, [base + 2 * lanes + 1])
    hs = []
    for j in range(4):
        src = sbuf.at[pl.ds(128 * j, 128)]
        hs.append(pltpu.async_copy(src, outs_hbm.at[idx_a_v.at[j]], sem))
        hs.append(pltpu.async_copy(src, outs_hbm.at[idx_b_v.at[j]], sem))
    for h in hs:
        h.wait()


_dispatch_index = pl.kernel(
    _index_body,
    out_type=[
        jax.ShapeDtypeStruct((_N,), jnp.int32),
        jax.ShapeDtypeStruct((_N, _SH), jnp.float32),
    ],
    mesh=plsc.VectorSubcoreMesh(core_axis_name="c", subcore_axis_name="s",
                                num_cores=_NC, num_subcores=_NS),
    scratch_types=[
        pltpu.VMEM((2 * _CHUNK,), jnp.int32),
        pltpu.VMEM((_CHUNK,), jnp.int32),
        pltpu.VMEM((_CHUNK,), jnp.int32),
        pltpu.VMEM((16,), jnp.int32),
        pltpu.VMEM((16,), jnp.int32),
        pltpu.VMEM((_NW * 16,), jnp.int32),
        pltpu.VMEM((16,), jnp.int32),
        pltpu.VMEM((_TOK, _SH), jnp.float32),
        pltpu.VMEM((4, 128), jnp.int32),
        pltpu.VMEM((4, 128), jnp.int32),
        pltpu.VMEM_SHARED((_NW * 16,), jnp.int32),
        pltpu.SemaphoreType.DMA,
    ],
    compiler_params=pltpu.CompilerParams(needs_layout_passes=False,
                                         use_tc_tiling_on_sc=False),
)


_W = 4096
_NWIN = _N // _W


def _move_body(x_hbm, dests_hbm, out_hbm,
               dwin0, dwin1, tok_v, buf, buf2, buf3,
               sem_d, sem_d2,
               sem_in, sem_in2, sem_in3, sem_out, sem_out2, sem_out3):
    c = lax.axis_index("c")
    s = lax.axis_index("s")
    wid = c * _NS + s
    lanes = lax.iota(jnp.int32, 16)
    lo = wid * _CHUNK

    dwins = (dwin0, dwin1)
    sems_d = (sem_d, sem_d2)
    h_d = [None, None]

    def start_d(win):
        i = win % 2
        h_d[i] = pltpu.async_copy(dests_hbm.at[pl.ds(win * _W, _W)],
                                  dwins[i], sems_d[i])

    start_d(0)
    for win in range(_NWIN):
        i = win % 2
        if win + 1 < _NWIN:
            start_d(win + 1)
        h_d[i].wait()

        def inv_step(r, _, i=i, win=win):
            d = dwins[i][pl.ds(r * 16, 16)]
            rel = d - lo
            m = jnp.logical_and(rel >= 0, rel < _CHUNK)
            src_tok = lax.shift_right_logical(win * _W + r * 16 + lanes, 1)
            plsc.store_scatter(tok_v, [jnp.where(m, rel, 0)], src_tok, mask=m)
            return 0

        lax.fori_loop(0, _W // 16, inv_step, 0)

    bufs = (buf, buf2, buf3)
    sems_in = (sem_in, sem_in2, sem_in3)
    sems_out = (sem_out, sem_out2, sem_out3)
    h_in = [None, None, None]
    h_out = [None, None, None]

    def start_in(b):
        i = b % 3
        idx = tok_v[pl.ds(b * _BT, _BT)]
        h_in[i] = pltpu.async_copy(x_hbm.at[idx], bufs[i], sems_in[i])

    start_in(0)
    start_in(1)
    for b in range(_NB):
        i = b % 3
        if b + 2 < _NB:
            j = (b + 2) % 3
            if h_out[j] is not None:
                h_out[j].wait()
            start_in(b + 2)
        h_in[i].wait()
        h_out[i] = pltpu.async_copy(
            bufs[i], out_hbm.at[pl.ds(lo + b * _BT, _BT)], sems_out[i])
    for h in h_out:
        if h is not None:
            h.wait()


_dispatch_move = pl.kernel(
    _move_body,
    out_type=[
        jax.ShapeDtypeStruct((_N, _H), jnp.float32),
    ],
    mesh=plsc.VectorSubcoreMesh(core_axis_name="c", subcore_axis_name="s",
                                num_cores=_NC, num_subcores=_NS),
    scratch_types=[
        pltpu.VMEM((_W,), jnp.int32),
        pltpu.VMEM((_W,), jnp.int32),
        pltpu.VMEM((_CHUNK,), jnp.int32),
        pltpu.VMEM((_BT, _H), jnp.float32),
        pltpu.VMEM((_BT, _H), jnp.float32),
        pltpu.VMEM((_BT, _H), jnp.float32),
        pltpu.SemaphoreType.DMA,
        pltpu.SemaphoreType.DMA,
        pltpu.SemaphoreType.DMA,
        pltpu.SemaphoreType.DMA,
        pltpu.SemaphoreType.DMA,
        pltpu.SemaphoreType.DMA,
        pltpu.SemaphoreType.DMA,
        pltpu.SemaphoreType.DMA,
    ],
    compiler_params=pltpu.CompilerParams(needs_layout_passes=False),
)


def kernel(total_token_num, expert_start_loc, recv_x, recv_x_scale, recv_topk,
           output_tensor, output_tensor_scale, output_index):
    del total_token_num, output_tensor, output_tensor_scale
    topk_flat = recv_topk.reshape(-1)
    oidx_flat, out_s = _dispatch_index(
        topk_flat, expert_start_loc.astype(jnp.int32), recv_x_scale)
    (out,) = _dispatch_move(recv_x, oidx_flat)
    return out, out_s, oidx_flat.reshape(output_index.shape)


# speedup vs baseline: 1.0243x; 1.0243x over previous
import jax
import jax.numpy as jnp
from jax import lax
from jax.experimental import pallas as pl
from jax.experimental.pallas import tpu as pltpu
from jax.experimental.pallas import tpu_sc as plsc

_T = 16384
_H = 2048
_SH = 16
_K = 2
_E = 16
_N = _T * _K
_NC = 2
_NS = 16
_NW = _NC * _NS
_CHUNK = _N // _NW
_TOK = _T // _NW
_BT = 16
_NB = _CHUNK // _BT
_VREGS = _CHUNK // 16


def _index_body(topk_hbm, starts_hbm, oidx_hbm,
                hist_ids_v, own_ids_v, dest_v, cnt_v, hist_v, hist_all_v,
                starts_v, hist_sh):
    c = lax.axis_index("c")
    s = lax.axis_index("s")
    wid = c * _NS + s
    lanes = lax.iota(jnp.int32, 16)

    pltpu.sync_copy(topk_hbm.at[pl.ds(s * 2 * _CHUNK, 2 * _CHUNK)], hist_ids_v)
    for j in range(2):
        def hist_step(r, acc, j=j):
            v = hist_ids_v[pl.ds(j * _CHUNK + r * 16, 16)]
            for e in range(_E):
                tot = jnp.sum((v == e).astype(jnp.int32))
                acc = jnp.where(lanes == e, acc + tot, acc)
            return acc
        hist_v[...] = lax.fori_loop(0, _VREGS, hist_step,
                                    jnp.zeros((16,), jnp.int32))
        pltpu.sync_copy(hist_v, hist_sh.at[pl.ds((2 * s + j) * 16, 16)])

    plsc.subcore_barrier()

    pltpu.sync_copy(hist_sh, hist_all_v)
    pltpu.sync_copy(starts_hbm, starts_v)
    base = lax.fori_loop(
        0, wid, lambda w, acc: acc + hist_all_v[pl.ds(w * 16, 16)],
        starts_v[...])
    cnt_v[...] = base

    pltpu.sync_copy(topk_hbm.at[pl.ds(wid * _CHUNK, _CHUNK)], own_ids_v)

    def rank_step(r, _):
        v = own_ids_v[pl.ds(r * 16, 16)]
        g = plsc.load_gather(cnt_v, [v])
        rank = jnp.zeros((16,), jnp.int32)
        vc = jnp.zeros((16,), jnp.int32)
        for e in range(_E):
            m = v == e
            cum = jnp.cumsum(m.astype(jnp.int32))
            rank = jnp.where(m, cum - 1, rank)
            tot = jnp.sum(m.astype(jnp.int32))
            vc = jnp.where(lanes == e, vc + tot, vc)
        dest_v[pl.ds(r * 16, 16)] = g + rank
        cnt_v[...] = cnt_v[...] + vc
        return 0

    lax.fori_loop(0, _VREGS, rank_step, 0)

    pltpu.sync_copy(dest_v, oidx_hbm.at[pl.ds(wid * _CHUNK, _CHUNK)])


_dispatch_index = pl.kernel(
    _index_body,
    out_type=[
        jax.ShapeDtypeStruct((_N,), jnp.int32),
    ],
    mesh=plsc.VectorSubcoreMesh(core_axis_name="c", subcore_axis_name="s",
                                num_cores=_NC, num_subcores=_NS),
    scratch_types=[
        pltpu.VMEM((2 * _CHUNK,), jnp.int32),
        pltpu.VMEM((_CHUNK,), jnp.int32),
        pltpu.VMEM((_CHUNK,), jnp.int32),
        pltpu.VMEM((16,), jnp.int32),
        pltpu.VMEM((16,), jnp.int32),
        pltpu.VMEM((_NW * 16,), jnp.int32),
        pltpu.VMEM((16,), jnp.int32),
        pltpu.VMEM_SHARED((_NW * 16,), jnp.int32),
    ],
    compiler_params=pltpu.CompilerParams(needs_layout_passes=False),
)


_W = 4096
_NWIN = _N // _W


def _move_body(x_hbm, dests_hbm, out_hbm,
               dwin0, dwin1, tok_v, buf, buf2, buf3,
               sem_d, sem_d2,
               sem_in, sem_in2, sem_in3, sem_out, sem_out2, sem_out3):
    c = lax.axis_index("c")
    s = lax.axis_index("s")
    wid = c * _NS + s
    lanes = lax.iota(jnp.int32, 16)
    lo = wid * _CHUNK

    dwins = (dwin0, dwin1)
    sems_d = (sem_d, sem_d2)
    h_d = [None, None]

    def start_d(win):
        i = win % 2
        h_d[i] = pltpu.async_copy(dests_hbm.at[pl.ds(win * _W, _W)],
                                  dwins[i], sems_d[i])

    start_d(0)
    for win in range(_NWIN):
        i = win % 2
        if win + 1 < _NWIN:
            start_d(win + 1)
        h_d[i].wait()

        def inv_step(r, _, i=i, win=win):
            d = dwins[i][pl.ds(r * 16, 16)]
            rel = d - lo
            m = jnp.logical_and(rel >= 0, rel < _CHUNK)
            src_tok = lax.shift_right_logical(win * _W + r * 16 + lanes, 1)
            plsc.store_scatter(tok_v, [jnp.where(m, rel, 0)], src_tok, mask=m)
            return 0

        lax.fori_loop(0, _W // 16, inv_step, 0)

    bufs = (buf, buf2, buf3)
    sems_in = (sem_in, sem_in2, sem_in3)
    sems_out = (sem_out, sem_out2, sem_out3)
    h_in = [None, None, None]
    h_out = [None, None, None]

    def start_in(b):
        i = b % 3
        idx = tok_v[pl.ds(b * _BT, _BT)]
        h_in[i] = pltpu.async_copy(x_hbm.at[idx], bufs[i], sems_in[i])

    start_in(0)
    start_in(1)
    for b in range(_NB):
        i = b % 3
        if b + 2 < _NB:
            j = (b + 2) % 3
            if h_out[j] is not None:
                h_out[j].wait()
            start_in(b + 2)
        h_in[i].wait()
        h_out[i] = pltpu.async_copy(
            bufs[i], out_hbm.at[pl.ds(lo + b * _BT, _BT)], sems_out[i])
    for h in h_out:
        if h is not None:
            h.wait()


_dispatch_move = pl.kernel(
    _move_body,
    out_type=[
        jax.ShapeDtypeStruct((_N, _H), jnp.float32),
    ],
    mesh=plsc.VectorSubcoreMesh(core_axis_name="c", subcore_axis_name="s",
                                num_cores=_NC, num_subcores=_NS),
    scratch_types=[
        pltpu.VMEM((_W,), jnp.int32),
        pltpu.VMEM((_W,), jnp.int32),
        pltpu.VMEM((_CHUNK,), jnp.int32),
        pltpu.VMEM((_BT, _H), jnp.float32),
        pltpu.VMEM((_BT, _H), jnp.float32),
        pltpu.VMEM((_BT, _H), jnp.float32),
        pltpu.SemaphoreType.DMA,
        pltpu.SemaphoreType.DMA,
        pltpu.SemaphoreType.DMA,
        pltpu.SemaphoreType.DMA,
        pltpu.SemaphoreType.DMA,
        pltpu.SemaphoreType.DMA,
        pltpu.SemaphoreType.DMA,
        pltpu.SemaphoreType.DMA,
    ],
    compiler_params=pltpu.CompilerParams(needs_layout_passes=False),
)


def _scale_body(xs_hbm, dests_hbm, outs_hbm,
                dest_v, sbuf, idx_a_v, idx_b_v, sem):
    c = lax.axis_index("c")
    s = lax.axis_index("s")
    wid = c * _NS + s
    lanes = lax.iota(jnp.int32, 16)

    pltpu.sync_copy(dests_hbm.at[pl.ds(wid * _CHUNK, _CHUNK)], dest_v)
    pltpu.sync_copy(xs_hbm.at[pl.ds(wid * _TOK, _TOK)], sbuf)

    for j in range(4):
        for i in range(8):
            base = 256 * j + 32 * i
            idx_a_v[j, pl.ds(16 * i, 16)] = plsc.load_gather(
                dest_v, [base + 2 * lanes])
            idx_b_v[j, pl.ds(16 * i, 16)] = plsc.load_gather(
                dest_v, [base + 2 * lanes + 1])

    hs = []
    for j in range(4):
        src = sbuf.at[pl.ds(128 * j, 128)]
        hs.append(pltpu.async_copy(src, outs_hbm.at[idx_a_v.at[j]], sem))
        hs.append(pltpu.async_copy(src, outs_hbm.at[idx_b_v.at[j]], sem))
    for h in hs:
        h.wait()


_dispatch_scale = pl.kernel(
    _scale_body,
    out_type=[
        jax.ShapeDtypeStruct((_N, _SH), jnp.float32),
    ],
    mesh=plsc.VectorSubcoreMesh(core_axis_name="c", subcore_axis_name="s",
                                num_cores=_NC, num_subcores=_NS),
    scratch_types=[
        pltpu.VMEM((_CHUNK,), jnp.int32),
        pltpu.VMEM((_TOK, _SH), jnp.float32),
        pltpu.VMEM((4, 128), jnp.int32),
        pltpu.VMEM((4, 128), jnp.int32),
        pltpu.SemaphoreType.DMA,
    ],
    compiler_params=pltpu.CompilerParams(needs_layout_passes=False,
                                         use_tc_tiling_on_sc=False),
)


def kernel(total_token_num, expert_start_loc, recv_x, recv_x_scale, recv_topk,
           output_tensor, output_tensor_scale, output_index):
    del total_token_num, output_tensor, output_tensor_scale
    topk_flat = recv_topk.reshape(-1)
    (oidx_flat,) = _dispatch_index(topk_flat,
                                   expert_start_loc.astype(jnp.int32))
    (out,) = _dispatch_move(recv_x, oidx_flat)
    (out_s,) = _dispatch_scale(recv_x_scale, oidx_flat)
    return out, out_s, oidx_flat.reshape(output_index.shape)


# speedup vs baseline: 1.0315x; 1.0070x over previous
import jax
import jax.numpy as jnp
from jax import lax
from jax.experimental import pallas as pl
from jax.experimental.pallas import tpu as pltpu
from jax.experimental.pallas import tpu_sc as plsc

_T = 16384
_H = 2048
_SH = 16
_K = 2
_E = 16
_N = _T * _K
_NC = 2
_NS = 16
_NW = _NC * _NS
_CHUNK = _N // _NW
_TOK = _T // _NW
_BT = 16
_NB = _CHUNK // _BT
_VREGS = _CHUNK // 16


def _index_body(topk_hbm, starts_hbm, oidx_hbm,
                hist_ids_v, own_ids_v, dest_v, cnt_v, hist_v, hist_all_v,
                starts_v, hist_sh):
    c = lax.axis_index("c")
    s = lax.axis_index("s")
    wid = c * _NS + s
    lanes = lax.iota(jnp.int32, 16)

    pltpu.sync_copy(topk_hbm.at[pl.ds(s * 2 * _CHUNK, 2 * _CHUNK)], hist_ids_v)
    ones = jnp.ones((16,), jnp.int32)
    for j in range(2):
        hist_v[...] = jnp.zeros((16,), jnp.int32)

        def hist_step(r, _, j=j):
            v = hist_ids_v[pl.ds(j * _CHUNK + r * 16, 16)]
            plsc.addupdate_scatter(hist_v, [v], ones)
            return 0

        lax.fori_loop(0, _VREGS, hist_step, 0)
        pltpu.sync_copy(hist_v, hist_sh.at[pl.ds((2 * s + j) * 16, 16)])

    plsc.subcore_barrier()

    pltpu.sync_copy(hist_sh, hist_all_v)
    pltpu.sync_copy(starts_hbm, starts_v)
    base = lax.fori_loop(
        0, wid, lambda w, acc: acc + hist_all_v[pl.ds(w * 16, 16)],
        starts_v[...])
    cnt_v[...] = base

    pltpu.sync_copy(topk_hbm.at[pl.ds(wid * _CHUNK, _CHUNK)], own_ids_v)

    def rank_step(r, _):
        v = own_ids_v[pl.ds(r * 16, 16)]
        g = plsc.load_gather(cnt_v, [v])
        sc, _unused = plsc.scan_count(v)
        dest_v[pl.ds(r * 16, 16)] = g + sc - 1
        plsc.addupdate_scatter(cnt_v, [v], ones)
        return 0

    lax.fori_loop(0, _VREGS, rank_step, 0)

    pltpu.sync_copy(dest_v, oidx_hbm.at[pl.ds(wid * _CHUNK, _CHUNK)])


_dispatch_index = pl.kernel(
    _index_body,
    out_type=[
        jax.ShapeDtypeStruct((_N,), jnp.int32),
    ],
    mesh=plsc.VectorSubcoreMesh(core_axis_name="c", subcore_axis_name="s",
                                num_cores=_NC, num_subcores=_NS),
    scratch_types=[
        pltpu.VMEM((2 * _CHUNK,), jnp.int32),
        pltpu.VMEM((_CHUNK,), jnp.int32),
        pltpu.VMEM((_CHUNK,), jnp.int32),
        pltpu.VMEM((16,), jnp.int32),
        pltpu.VMEM((16,), jnp.int32),
        pltpu.VMEM((_NW * 16,), jnp.int32),
        pltpu.VMEM((16,), jnp.int32),
        pltpu.VMEM_SHARED((_NW * 16,), jnp.int32),
    ],
    compiler_params=pltpu.CompilerParams(needs_layout_passes=False),
)


_W = 4096
_NWIN = _N // _W


def _move_body(x_hbm, dests_hbm, out_hbm,
               dwin0, dwin1, tok_v, buf, buf2, buf3,
               sem_d, sem_d2,
               sem_in, sem_in2, sem_in3, sem_out, sem_out2, sem_out3):
    c = lax.axis_index("c")
    s = lax.axis_index("s")
    wid = c * _NS + s
    lanes = lax.iota(jnp.int32, 16)
    lo = wid * _CHUNK

    dwins = (dwin0, dwin1)
    sems_d = (sem_d, sem_d2)
    h_d = [None, None]

    def start_d(win):
        i = win % 2
        h_d[i] = pltpu.async_copy(dests_hbm.at[pl.ds(win * _W, _W)],
                                  dwins[i], sems_d[i])

    start_d(0)
    for win in range(_NWIN):
        i = win % 2
        if win + 1 < _NWIN:
            start_d(win + 1)
        h_d[i].wait()

        def inv_step(r, _, i=i, win=win):
            d = dwins[i][pl.ds(r * 16, 16)]
            rel = d - lo
            m = jnp.logical_and(rel >= 0, rel < _CHUNK)
            src_tok = lax.shift_right_logical(win * _W + r * 16 + lanes, 1)
            plsc.store_scatter(tok_v, [jnp.where(m, rel, 0)], src_tok, mask=m)
            return 0

        lax.fori_loop(0, _W // 16, inv_step, 0)

    bufs = (buf, buf2, buf3)
    sems_in = (sem_in, sem_in2, sem_in3)
    sems_out = (sem_out, sem_out2, sem_out3)
    h_in = [None, None, None]
    h_out = [None, None, None]

    def start_in(b):
        i = b % 3
        idx = tok_v[pl.ds(b * _BT, _BT)]
        h_in[i] = pltpu.async_copy(x_hbm.at[idx], bufs[i], sems_in[i])

    start_in(0)
    start_in(1)
    for b in range(_NB):
        i = b % 3
        if b + 2 < _NB:
            j = (b + 2) % 3
            if h_out[j] is not None:
                h_out[j].wait()
            start_in(b + 2)
        h_in[i].wait()
        h_out[i] = pltpu.async_copy(
            bufs[i], out_hbm.at[pl.ds(lo + b * _BT, _BT)], sems_out[i])
    for h in h_out:
        if h is not None:
            h.wait()


_dispatch_move = pl.kernel(
    _move_body,
    out_type=[
        jax.ShapeDtypeStruct((_N, _H), jnp.float32),
    ],
    mesh=plsc.VectorSubcoreMesh(core_axis_name="c", subcore_axis_name="s",
                                num_cores=_NC, num_subcores=_NS),
    scratch_types=[
        pltpu.VMEM((_W,), jnp.int32),
        pltpu.VMEM((_W,), jnp.int32),
        pltpu.VMEM((_CHUNK,), jnp.int32),
        pltpu.VMEM((_BT, _H), jnp.float32),
        pltpu.VMEM((_BT, _H), jnp.float32),
        pltpu.VMEM((_BT, _H), jnp.float32),
        pltpu.SemaphoreType.DMA,
        pltpu.SemaphoreType.DMA,
        pltpu.SemaphoreType.DMA,
        pltpu.SemaphoreType.DMA,
        pltpu.SemaphoreType.DMA,
        pltpu.SemaphoreType.DMA,
        pltpu.SemaphoreType.DMA,
        pltpu.SemaphoreType.DMA,
    ],
    compiler_params=pltpu.CompilerParams(needs_layout_passes=False),
)


def _scale_body(xs_hbm, dests_hbm, outs_hbm,
                dest_v, sbuf, idx_a_v, idx_b_v, sem):
    c = lax.axis_index("c")
    s = lax.axis_index("s")
    wid = c * _NS + s
    lanes = lax.iota(jnp.int32, 16)

    pltpu.sync_copy(dests_hbm.at[pl.ds(wid * _CHUNK, _CHUNK)], dest_v)
    pltpu.sync_copy(xs_hbm.at[pl.ds(wid * _TOK, _TOK)], sbuf)

    for j in range(4):
        for i in range(8):
            base = 256 * j + 32 * i
            idx_a_v[j, pl.ds(16 * i, 16)] = plsc.load_gather(
                dest_v, [base + 2 * lanes])
            idx_b_v[j, pl.ds(16 * i, 16)] = plsc.load_gather(
                dest_v, [base + 2 * lanes + 1])

    hs = []
    for j in range(4):
        src = sbuf.at[pl.ds(128 * j, 128)]
        hs.append(pltpu.async_copy(src, outs_hbm.at[idx_a_v.at[j]], sem))
        hs.append(pltpu.async_copy(src, outs_hbm.at[idx_b_v.at[j]], sem))
    for h in hs:
        h.wait()


_dispatch_scale = pl.kernel(
    _scale_body,
    out_type=[
        jax.ShapeDtypeStruct((_N, _SH), jnp.float32),
    ],
    mesh=plsc.VectorSubcoreMesh(core_axis_name="c", subcore_axis_name="s",
                                num_cores=_NC, num_subcores=_NS),
    scratch_types=[
        pltpu.VMEM((_CHUNK,), jnp.int32),
        pltpu.VMEM((_TOK, _SH), jnp.float32),
        pltpu.VMEM((4, 128), jnp.int32),
        pltpu.VMEM((4, 128), jnp.int32),
        pltpu.SemaphoreType.DMA,
    ],
    compiler_params=pltpu.CompilerParams(needs_layout_passes=False,
                                         use_tc_tiling_on_sc=False),
)


def kernel(total_token_num, expert_start_loc, recv_x, recv_x_scale, recv_topk,
           output_tensor, output_tensor_scale, output_index):
    del total_token_num, output_tensor, output_tensor_scale
    topk_flat = recv_topk.reshape(-1)
    (oidx_flat,) = _dispatch_index(topk_flat,
                                   expert_start_loc.astype(jnp.int32))
    (out,) = _dispatch_move(recv_x, oidx_flat)
    (out_s,) = _dispatch_scale(recv_x_scale, oidx_flat)
    return out, out_s, oidx_flat.reshape(output_index.shape)
